# baseline (speedup 1.0000x reference)
import functools

import jax
import jax.numpy as jnp
from jax import lax
from jax.experimental import pallas as pl
from jax.experimental.pallas import tpu as pltpu
from jax.experimental.pallas import tpu_sc as plsc
from jax._src.pallas import mpmd as _mpmd

_N_TRACKS = 100000
_Q = 3
_N = 128
_B = 4096
_NC, _NS, _L = 1, 16, 16
_NW = _NC * _NS
_TRW = (_N_TRACKS // _NW + 15) & ~7
_TRP = (_TRW + 15) & ~15
_CAP = _B + 2 * _L
_NVEC = _B // _L
_TRS = _TRP
_NTP = (_NW - 1) * _TRW + _TRS
_I32 = jnp.int32
_F32 = jnp.float32


def _splat(x):
    return jnp.broadcast_to(x, (_L,))


def _dot(a, b):
    p = [x * y for x, y in zip(a, b)]
    s01, s23 = p[0] + p[1], p[2] + p[3]
    s45, s67 = p[4] + p[5], p[6] + p[7]
    s = (s01 + s23) + (s45 + s67)
    return _splat(jnp.sum(s))


def _rsqrt(x):
    i = plsc.bitcast(x, _I32)
    y = plsc.bitcast(jnp.int32(0x5F3759DF) - (i >> 1), _F32)
    for _ in range(4):
        y = y * (1.5 - 0.5 * x * y * y)
    return y


def _body(reprs_hbm, idx_hbm, aux_hbm, mem_in, mem_out,
          idx_v, pos1, trk1, pos2, trk2, table,
          mbuf, rbuf, auxbuf, abuf, sem_m, sem_r, sem_s):
    wid = lax.axis_index("s") * _NC + lax.axis_index("c")
    base = pl.multiple_of(wid * _TRW, 8)
    lanes = lax.iota(_I32, _L)

    pltpu.sync_copy(idx_hbm, idx_v)
    for p in range(4):
        pltpu.sync_copy(
            aux_hbm.at[pl.ds(pl.multiple_of(p * _NTP + base, 8), _TRS)],
            auxbuf.at[pl.ds(p * _TRS, _TRS)])

    def init_tab(v, c):
        table[pl.ds(v * _L, _L)] = jnp.full((_L,), -1, _I32)
        return c
    lax.fori_loop(0, _TRP // _L, init_tab, 0)

    def init_trk(v, c):
        trk1[pl.ds(v * _L, _L)] = jnp.zeros((_L,), _I32)
        return c
    lax.fori_loop(0, _CAP // _L, init_trk, 0)

    def sel(v, cnt):
        tvec = idx_v[pl.ds(pl.multiple_of(v * _L, _L), _L)]
        m = (tvec >= base) & (tvec < base + _TRW)
        mi = m.astype(_I32)
        tgt = cnt + plsc.cumsum(mi) - mi
        plsc.store_scatter(pos1, [tgt], v * _L + lanes, mask=m)
        plsc.store_scatter(trk1, [tgt], tvec - base, mask=m)
        return cnt + jnp.sum(mi)
    k = lax.fori_loop(0, _NVEC, sel, 0)

    nv1 = (k + _L - 1) >> 4

    def wr(v, c):
        tvec = trk1[pl.ds(pl.multiple_of(v * _L, _L), _L)]
        jvec = v * _L + lanes
        live = jvec < k
        for i in range(_L):
            plsc.store_scatter(table, [tvec], jvec,
                               mask=live & (lanes == i))
        return c
    lax.fori_loop(0, nv1, wr, 0)

    def win_sel(v, cnt):
        tvec = trk1[pl.ds(pl.multiple_of(v * _L, _L), _L)]
        jvec = v * _L + lanes
        w = plsc.load_gather(table, [tvec]) == jvec
        wi = w.astype(_I32)
        tgt = cnt + plsc.cumsum(wi) - wi
        plsc.store_scatter(pos2, [tgt],
                           pos1[pl.ds(pl.multiple_of(v * _L, _L), _L)], mask=w)
        plsc.store_scatter(trk2, [tgt], tvec, mask=w)
        return cnt + jnp.sum(wi)
    k2 = lax.fori_loop(0, nv1, win_sel, 0)

    kp = ((k2 + _L - 1) >> 4) << 4

    @pl.when(k2 > 0)
    def _():
        last = jnp.full((_L,), k2 - 1, _I32)
        plsc.store_scatter(pos2, [k2 + lanes], plsc.load_gather(pos2, [last]))
        plsc.store_scatter(trk2, [k2 + lanes], plsc.load_gather(trk2, [last]))

    def chunk(c, carry):
        gvec = trk2[pl.ds(pl.multiple_of(c * _L, _L), _L)] + base
        pvec = pos2[pl.ds(pl.multiple_of(c * _L, _L), _L)]
        cms = [pltpu.async_copy(mem_in.at[gvec + (q * _N_TRACKS)],
                                mbuf.at[q], sem_m) for q in range(_Q)]
        cr = pltpu.async_copy(reprs_hbm.at[pvec], rbuf, sem_r)
        tloc = gvec - base
        for p in range(4):
            abuf[pl.ds(p * _L, _L)] = plsc.load_gather(
                auxbuf, [tloc + (p * _TRS)])
        for cm in cms:
            cm.wait()
        cr.wait()

        def item(i, cc):
            iv = jnp.full((_L,), i, _I32)
            r = [rbuf[i, pl.ds(16 * j, _L)] for j in range(8)]
            rs = _rsqrt(_dot(r, r))
            r = [x * rs for x in r]
            cv = plsc.load_gather(abuf, [iv + (3 * _L)])
            prev = None
            for q in range(_Q):
                mq = [mbuf[q, i, pl.ds(16 * j, _L)] for j in range(8)]
                bq0 = plsc.load_gather(abuf, [iv + (q * _L)])
                bnew = _dot(r, mq)
                bnew = jnp.where(cv == 0.0, 0.1, bnew)
                bq = (bq0 * cv + bnew) / (cv + 1.0)
                bq = jnp.clip(bq, 0.1, 0.9)
                if q == 0:
                    alpha = jnp.full((_L,), 0.1, _F32)
                else:
                    alpha = jnp.clip(_dot(mq, prev), 0.1, 0.9)
                c1 = 1.0 - (alpha + bq) * (1.0 / 3.0)
                c2 = alpha * (1.0 / 3.0)
                c3 = bq * (1.0 / 3.0)
                if q == 0:
                    m = [c1 * a + c3 * b for a, b in zip(mq, r)]
                else:
                    m = [c1 * a + c2 * p + c3 * b
                         for a, p, b in zip(mq, prev, r)]
                ms = _rsqrt(_dot(m, m))
                m = [x * ms for x in m]
                for j in range(8):
                    mbuf[q, i, pl.ds(16 * j, _L)] = m[j]
                prev = m
            return cc
        lax.fori_loop(0, _L, item, 0)
        css = [pltpu.async_copy(mbuf.at[q], mem_out.at[gvec + (q * _N_TRACKS)],
                                sem_s) for q in range(_Q)]
        for cs in css:
            cs.wait()
        return carry
    lax.fori_loop(0, kp >> 4, chunk, 0)


@functools.lru_cache(maxsize=1)
def _build():
    mesh = plsc.VectorSubcoreMesh(core_axis_name="c", subcore_axis_name="s",
                                  num_cores=_NC, num_subcores=_NS)
    return _mpmd._mpmd_map(
        [(mesh, _body)],
        jax.ShapeDtypeStruct((_Q * _N_TRACKS, _N), _F32),
        input_output_aliases={3: 0},
        compiler_params=pltpu.CompilerParams(needs_layout_passes=False),
        scratch_types=[
            pltpu.VMEM((_B,), _I32),
            pltpu.VMEM((_CAP,), _I32),
            pltpu.VMEM((_CAP,), _I32),
            pltpu.VMEM((_CAP,), _I32),
            pltpu.VMEM((_CAP,), _I32),
            pltpu.VMEM((_TRP,), _I32),
            pltpu.VMEM((_Q, _L, _N), _F32),
            pltpu.VMEM((_L, _N), _F32),
            pltpu.VMEM((4 * _TRS,), _F32),
            pltpu.VMEM((4 * _L,), _F32),
            pltpu.SemaphoreType.DMA,
            pltpu.SemaphoreType.DMA,
            pltpu.SemaphoreType.DMA,
        ],
        name="memory_bank_infer_sc",
    )


def kernel(reprs, track_idxs, beta, count, memory):
    idx = track_idxs.astype(_I32)
    aux = jnp.concatenate([beta.T, count.T], axis=0)
    aux = jnp.concatenate(
        [aux, jnp.zeros((4, _NTP - _N_TRACKS), _F32)], axis=1).reshape(-1)
    mem_flat = memory.transpose(1, 0, 2).reshape(_Q * _N_TRACKS, _N)
    out = _build()(reprs, idx, aux, mem_flat)
    return out.reshape(_Q, _N_TRACKS, _N).transpose(1, 0, 2)


# speedup vs baseline: 1.2823x; 1.2823x over previous
import functools

import jax
import jax.numpy as jnp
from jax import lax
from jax.experimental import pallas as pl
from jax.experimental.pallas import tpu as pltpu
from jax.experimental.pallas import tpu_sc as plsc
from jax._src.pallas import mpmd as _mpmd

_N_TRACKS = 100000
_Q = 3
_N = 128
_B = 4096
_NC, _NS, _L = 2, 16, 16
_NW = _NC * _NS
_TRW = 3128
_TRL = _N_TRACKS - (_NW - 1) * _TRW
_TRP = 3136
_CAP = _B + 2 * _L
_NVEC = _B // _L
_TRS = 3136
_NTP = (_NW - 1) * _TRW + _TRS
_I32 = jnp.int32
_F32 = jnp.float32


def _splat(x):
    return jnp.broadcast_to(x, (_L,))


def _dot(a, b):
    p = [x * y for x, y in zip(a, b)]
    s01, s23 = p[0] + p[1], p[2] + p[3]
    s45, s67 = p[4] + p[5], p[6] + p[7]
    s = (s01 + s23) + (s45 + s67)
    return _splat(jnp.sum(s))


def _rsqrt(x):
    i = plsc.bitcast(x, _I32)
    y = plsc.bitcast(jnp.int32(0x5F3759DF) - (i >> 1), _F32)
    for _ in range(4):
        y = y * (1.5 - 0.5 * x * y * y)
    return y


def _body(reprs_hbm, idx_hbm, aux_hbm, mem_in, mem_out,
          idx_v, pos1, trk1, pos2, trk2, table,
          mbuf, rbuf, auxbuf, abuf,
          sem_m0, sem_m1, sem_r0, sem_r1, sem_s0, sem_s1):
    wid = lax.axis_index("s") * _NC + lax.axis_index("c")
    base = pl.multiple_of(wid * _TRW, 8)
    lanes = lax.iota(_I32, _L)

    pltpu.sync_copy(idx_hbm, idx_v)
    for p in range(4):
        pltpu.sync_copy(
            aux_hbm.at[pl.ds(pl.multiple_of(p * _NTP + base, 8), _TRS)],
            auxbuf.at[pl.ds(p * _TRS, _TRS)])

    def init_tab(v, c):
        table[pl.ds(v * _L, _L)] = jnp.full((_L,), -1, _I32)
        return c
    lax.fori_loop(0, _TRP // _L, init_tab, 0)

    def init_trk(v, c):
        trk1[pl.ds(v * _L, _L)] = jnp.zeros((_L,), _I32)
        return c
    lax.fori_loop(0, _CAP // _L, init_trk, 0)

    def sel(v, cnt):
        tvec = idx_v[pl.ds(pl.multiple_of(v * _L, _L), _L)]
        m = (tvec >= base) & (tvec < base + _TRW)
        mi = m.astype(_I32)
        tgt = cnt + plsc.cumsum(mi) - mi
        plsc.store_scatter(pos1, [tgt], v * _L + lanes, mask=m)
        plsc.store_scatter(trk1, [tgt], tvec - base, mask=m)
        return cnt + jnp.sum(mi)
    k = lax.fori_loop(0, _NVEC, sel, 0)

    nv1 = (k + _L - 1) >> 4

    def wr(v, c):
        tvec = trk1[pl.ds(pl.multiple_of(v * _L, _L), _L)]
        jvec = v * _L + lanes
        live = jvec < k
        for i in range(_L):
            plsc.store_scatter(table, [tvec], jvec,
                               mask=live & (lanes == i))
        return c
    lax.fori_loop(0, nv1, wr, 0)

    def win_sel(v, cnt):
        tvec = trk1[pl.ds(pl.multiple_of(v * _L, _L), _L)]
        jvec = v * _L + lanes
        w = plsc.load_gather(table, [tvec]) == jvec
        wi = w.astype(_I32)
        tgt = cnt + plsc.cumsum(wi) - wi
        plsc.store_scatter(pos2, [tgt],
                           pos1[pl.ds(pl.multiple_of(v * _L, _L), _L)], mask=w)
        plsc.store_scatter(trk2, [tgt], tvec, mask=w)
        return cnt + jnp.sum(wi)
    k2 = lax.fori_loop(0, nv1, win_sel, 0)

    kp = ((k2 + _L - 1) >> 4) << 4

    @pl.when(k2 > 0)
    def _():
        last = jnp.full((_L,), k2 - 1, _I32)
        plsc.store_scatter(pos2, [k2 + lanes], plsc.load_gather(pos2, [last]))
        plsc.store_scatter(trk2, [k2 + lanes], plsc.load_gather(trk2, [last]))

    nch = kp >> 4
    sems_m = (sem_m0, sem_m1)
    sems_r = (sem_r0, sem_r1)
    sems_s = (sem_s0, sem_s1)

    def issue_gathers(c, par):
        gvec = trk2[pl.ds(pl.multiple_of(c * _L, _L), _L)] + base
        pvec = pos2[pl.ds(pl.multiple_of(c * _L, _L), _L)]
        for q in range(_Q):
            pltpu.make_async_copy(mem_in.at[gvec + (q * _N_TRACKS)],
                                  mbuf.at[par, q], sems_m[par]).start()
        pltpu.make_async_copy(reprs_hbm.at[pvec], rbuf.at[par],
                              sems_r[par]).start()

    def drain(dst_ref, sem):
        pltpu.make_async_copy(mem_in.at[pl.ds(0, _L)], dst_ref, sem).wait()

    @pl.when(nch > 0)
    def _():
        issue_gathers(0, 0)

    def chunk_pair(pi, carry):
        for par in (0, 1):
            c = pi * 2 + par

            @pl.when(c < nch)
            def _(par=par, c=c):
                @pl.when(c + 1 < nch)
                def _(par=par, c=c):
                    @pl.when(c >= 1)
                    def _(par=par):
                        for q in range(_Q):
                            drain(mbuf.at[1 - par, q], sems_s[1 - par])
                    issue_gathers(c + 1, 1 - par)
                for q in range(_Q):
                    drain(mbuf.at[par, q], sems_m[par])
                drain(rbuf.at[par], sems_r[par])

                gvec = trk2[pl.ds(pl.multiple_of(c * _L, _L), _L)] + base
                tloc = gvec - base
                for p in range(4):
                    abuf[pl.ds(p * _L, _L)] = plsc.load_gather(
                        auxbuf, [tloc + (p * _TRS)])

                def item(i, cc):
                    iv = jnp.full((_L,), i, _I32)
                    r = [rbuf[par, i, pl.ds(16 * j, _L)] for j in range(8)]
                    rs = _rsqrt(_dot(r, r))
                    r = [x * rs for x in r]
                    cv = plsc.load_gather(abuf, [iv + (3 * _L)])
                    prev = None
                    for q in range(_Q):
                        mq = [mbuf[par, q, i, pl.ds(16 * j, _L)]
                              for j in range(8)]
                        bq0 = plsc.load_gather(abuf, [iv + (q * _L)])
                        bnew = _dot(r, mq)
                        bnew = jnp.where(cv == 0.0, 0.1, bnew)
                        bq = (bq0 * cv + bnew) / (cv + 1.0)
                        bq = jnp.clip(bq, 0.1, 0.9)
                        if q == 0:
                            alpha = jnp.full((_L,), 0.1, _F32)
                        else:
                            alpha = jnp.clip(_dot(mq, prev), 0.1, 0.9)
                        c1 = 1.0 - (alpha + bq) * (1.0 / 3.0)
                        c2 = alpha * (1.0 / 3.0)
                        c3 = bq * (1.0 / 3.0)
                        if q == 0:
                            m = [c1 * a + c3 * b for a, b in zip(mq, r)]
                        else:
                            m = [c1 * a + c2 * p + c3 * b
                                 for a, p, b in zip(mq, prev, r)]
                        ms = _rsqrt(_dot(m, m))
                        m = [x * ms for x in m]
                        for j in range(8):
                            mbuf[par, q, i, pl.ds(16 * j, _L)] = m[j]
                        prev = m
                    return cc
                lax.fori_loop(0, _L, item, 0)
                for q in range(_Q):
                    pltpu.make_async_copy(
                        mbuf.at[par, q], mem_out.at[gvec + (q * _N_TRACKS)],
                        sems_s[par]).start()
        return carry
    lax.fori_loop(0, (nch + 1) >> 1, chunk_pair, 0)

    @pl.when(nch >= 1)
    def _():
        for q in range(_Q):
            drain(mbuf.at[0, q], sems_s[0])

    @pl.when(nch >= 2)
    def _():
        for q in range(_Q):
            drain(mbuf.at[1, q], sems_s[1])


@functools.lru_cache(maxsize=1)
def _build():
    mesh = plsc.VectorSubcoreMesh(core_axis_name="c", subcore_axis_name="s",
                                  num_cores=_NC, num_subcores=_NS)
    return _mpmd._mpmd_map(
        [(mesh, _body)],
        jax.ShapeDtypeStruct((_Q * _N_TRACKS, _N), _F32),
        input_output_aliases={3: 0},
        compiler_params=pltpu.CompilerParams(needs_layout_passes=False),
        scratch_types=[
            pltpu.VMEM((_B,), _I32),
            pltpu.VMEM((_CAP,), _I32),
            pltpu.VMEM((_CAP,), _I32),
            pltpu.VMEM((_CAP,), _I32),
            pltpu.VMEM((_CAP,), _I32),
            pltpu.VMEM((_TRP,), _I32),
            pltpu.VMEM((2, _Q, _L, _N), _F32),
            pltpu.VMEM((2, _L, _N), _F32),
            pltpu.VMEM((4 * _TRS,), _F32),
            pltpu.VMEM((4 * _L,), _F32),
            pltpu.SemaphoreType.DMA,
            pltpu.SemaphoreType.DMA,
            pltpu.SemaphoreType.DMA,
            pltpu.SemaphoreType.DMA,
            pltpu.SemaphoreType.DMA,
            pltpu.SemaphoreType.DMA,
        ],
        name="memory_bank_infer_sc",
    )


def kernel(reprs, track_idxs, beta, count, memory):
    idx = track_idxs.astype(_I32)
    aux = jnp.concatenate([beta.T, count.T], axis=0)
    aux = jnp.concatenate(
        [aux, jnp.zeros((4, _NTP - _N_TRACKS), _F32)], axis=1).reshape(-1)
    mem_flat = memory.transpose(1, 0, 2).reshape(_Q * _N_TRACKS, _N)
    out = _build()(reprs, idx, aux, mem_flat)
    return out.reshape(_Q, _N_TRACKS, _N).transpose(1, 0, 2)
